# baseline (speedup 1.0000x reference)
import functools

import jax
import jax.numpy as jnp
from jax import lax
from jax.experimental import pallas as pl
from jax.experimental.pallas import tpu as pltpu
from jax.experimental.pallas import tpu_sc as plsc

N = 1 << 22
K = 100000
NW = 32
PW = N // NW
CHUNK = 16384
NCHUNK = PW // CHUNK
VPC = CHUNK // 16
NB0 = 2048
NB1 = 2048
NB2 = 1024
SCALE = float(K) * float(N)


def _sc_mesh():
    return plsc.VectorSubcoreMesh(core_axis_name="c", subcore_axis_name="s")


def _zero_hist(ref, nwords):
    zeros = jnp.zeros((16,), jnp.float32)

    @plsc.parallel_loop(0, nwords // 16, unroll=8)
    def _(i):
        ref[pl.ds(i * 16, 16)] = zeros


def _hist_body(level):
    nb = (NB0, NB1, NB2)[level]
    sums = level < 2

    def body(*args):
        if level == 0:
            u_hbm = args[0]
            sel_hbm = None
            rest = args[1:]
        else:
            u_hbm, sel_hbm = args[0], args[1]
            rest = args[2:]
        if sums:
            cnt_hbm, sum_hbm = rest[0], rest[1]
            uv0, uv1, selv, cntv, sumv, sem0, sem1 = rest[2:]
        else:
            cnt_hbm = rest[0]
            sum_hbm = None
            uv0, uv1, selv, cntv, sumv, sem0, sem1 = rest[1:]

        wid = lax.axis_index("c") * 16 + lax.axis_index("s")
        base = wid * PW
        li = lax.iota(jnp.int32, 16)
        ones = jnp.ones((16,), jnp.float32)

        _zero_hist(cntv, nb * 16)
        if sums:
            _zero_hist(sumv, nb * 16)
        if level > 0:
            pltpu.sync_copy(sel_hbm.at[pl.ds(0, 16)], selv)
            selvec = selv[...]

        sems = (sem0, sem1)
        bufs = (uv0, uv1)
        pltpu.async_copy(u_hbm.at[pl.ds(base, CHUNK)], bufs[0], sems[0])
        for ci in range(NCHUNK):
            cur = ci % 2
            if ci + 1 < NCHUNK:
                pltpu.async_copy(
                    u_hbm.at[pl.ds(base + (ci + 1) * CHUNK, CHUNK)],
                    bufs[(ci + 1) % 2], sems[(ci + 1) % 2])
            pltpu.make_async_copy(
                u_hbm.at[pl.ds(base + ci * CHUNK, CHUNK)], bufs[cur],
                sems[cur]).wait()
            buf = bufs[cur]

            @plsc.parallel_loop(0, VPC, unroll=16)
            def _(vi):
                u = buf[pl.ds(vi * 16, 16)]
                bits = lax.bitcast_convert_type(u, jnp.int32)
                if level == 0:
                    flat = lax.shift_right_logical(bits, 21) * 16 + li
                    plsc.addupdate_scatter(cntv, [flat], ones)
                    plsc.addupdate_scatter(sumv, [flat], u)
                elif level == 1:
                    m = lax.shift_right_logical(bits, 21) == selvec

                    @pl.when(jnp.any(m))
                    def _():
                        sub = lax.shift_right_logical(bits, 10) & 0x7FF
                        flat = sub * 16 + li
                        plsc.addupdate_scatter(cntv, [flat], ones, mask=m)
                        plsc.addupdate_scatter(sumv, [flat], u, mask=m)
                else:
                    m = lax.shift_right_logical(bits, 10) == selvec

                    @pl.when(jnp.any(m))
                    def _():
                        flat = (bits & 0x3FF) * 16 + li
                        plsc.addupdate_scatter(cntv, [flat], ones, mask=m)

        pltpu.sync_copy(cntv, cnt_hbm.at[wid])
        if sums:
            pltpu.sync_copy(sumv, sum_hbm.at[wid])

    return body


def _sc_hist(level):
    nb = (NB0, NB1, NB2)[level]
    sums = level < 2
    f32 = jnp.float32
    hist = jax.ShapeDtypeStruct((NW, nb * 16), f32)
    out_type = (hist, hist) if sums else hist
    return pl.kernel(
        _hist_body(level),
        out_type=out_type,
        mesh=_sc_mesh(),
        compiler_params=pltpu.CompilerParams(needs_layout_passes=False),
        scratch_types=[
            pltpu.VMEM((CHUNK,), f32),
            pltpu.VMEM((CHUNK,), f32),
            pltpu.VMEM((16,), jnp.int32),
            pltpu.VMEM((nb * 16,), f32),
            pltpu.VMEM((nb * 16 if sums else 16,), f32),
            pltpu.SemaphoreType.DMA,
            pltpu.SemaphoreType.DMA,
        ],
    )


def _tc_elem(p_ref, t_ref, w_ref, u_ref):
    d = p_ref[...] - t_ref[...]
    u2 = w_ref[...] * d * d
    for j in range(4):
        u_ref[pl.ds(j * 512, 512), :] = u2[:, j * 128:(j + 1) * 128]


def _cell_select(cnt, kk):
    R, C = cnt.shape
    ci = lax.broadcasted_iota(jnp.int32, (C, C), 0)
    cj = lax.broadcasted_iota(jnp.int32, (C, C), 1)
    mg = ((ci // 16) >= (cj // 16)).astype(jnp.float32)
    gs = jnp.dot(cnt, mg, preferred_element_type=jnp.float32)
    rt = jnp.sum(cnt, axis=1, keepdims=True)
    ri = lax.broadcasted_iota(jnp.int32, (R, R), 0)
    rj = lax.broadcasted_iota(jnp.int32, (R, R), 1)
    mr = (rj > ri).astype(jnp.float32)
    sr = jnp.dot(mr, rt, preferred_element_type=jnp.float32)
    incl = sr + gs
    bidx = (lax.broadcasted_iota(jnp.int32, (R, C), 0) * 8
            + lax.broadcasted_iota(jnp.int32, (R, C), 1) // 16)
    sel = jnp.max(jnp.where(incl >= kk, bidx, -1))
    above = bidx > sel
    return sel, above, bidx


def _worker_sum(ref, rows):
    acc = ref[pl.ds(0, rows), :]
    for w in range(1, NW):
        acc = acc + ref[pl.ds(w * rows, rows), :]
    return acc


def _scalar_at(ref_val, r, c):
    row = lax.broadcasted_iota(jnp.int32, ref_val.shape, 0)
    col = lax.broadcasted_iota(jnp.int32, ref_val.shape, 1)
    zero = jnp.zeros((), ref_val.dtype)
    return jnp.sum(jnp.where((row == r) & (col == c), ref_val, zero))


def _bcast_rows(vals, dtype):
    out = jnp.zeros((8, 128), dtype)
    row = lax.broadcasted_iota(jnp.int32, (8, 128), 0)
    for r, v in enumerate(vals):
        out = jnp.where(row == r, v.astype(dtype), out)
    return out


def _tc_m1(cnt_ref, sum_ref, sel_out, st_out):
    cnt = _worker_sum(cnt_ref, 256)
    sm = _worker_sum(sum_ref, 256)
    kk = jnp.float32(K)
    sel, above, _ = _cell_select(cnt, kk)
    c_above = jnp.sum(jnp.where(above, cnt, 0.0))
    s_above = jnp.sum(jnp.where(above, sm, 0.0))
    sel_out[...] = jnp.full((8, 128), sel, jnp.int32)
    st_out[...] = _bcast_rows([kk - c_above, s_above], jnp.float32)


def _tc_m2(cnt_ref, sum_ref, sel_ref, st_ref, sel_out, st_out):
    cnt = _worker_sum(cnt_ref, 256)
    sm = _worker_sum(sum_ref, 256)
    b0 = _scalar_at(sel_ref[...], 0, 0)
    st = st_ref[...]
    k1 = _scalar_at(st, 0, 0)
    s0 = _scalar_at(st, 1, 0)
    sel, above, _ = _cell_select(cnt, k1)
    c_above = jnp.sum(jnp.where(above, cnt, 0.0))
    s_above = jnp.sum(jnp.where(above, sm, 0.0))
    prefix22 = b0 * NB1 + sel
    sel_out[...] = jnp.full((8, 128), prefix22, jnp.int32)
    st_out[...] = _bcast_rows([k1 - c_above, s0 + s_above], jnp.float32)


def _tc_m3(cnt_ref, sel_ref, st_ref, ans_out):
    cnt = _worker_sum(cnt_ref, 128)
    prefix22 = _scalar_at(sel_ref[...], 0, 0)
    st = st_ref[...]
    k2 = _scalar_at(st, 0, 0)
    s01 = _scalar_at(st, 1, 0)
    sel, above, bidx = _cell_select(cnt, k2)
    c_above = jnp.sum(jnp.where(above, cnt, 0.0))
    vals = lax.bitcast_convert_type(prefix22 * NB2 + bidx, jnp.float32)
    s2 = jnp.sum(jnp.where(above, cnt * vals, 0.0))
    t = lax.bitcast_convert_type(prefix22 * NB2 + sel, jnp.float32)
    ans = (s01 + s2 + (k2 - c_above) * t) / jnp.float32(SCALE)
    ans_out[...] = jnp.full((1, 1), ans, jnp.float32)


def kernel(predict, target, weight):
    f32 = jnp.float32
    p2 = predict.reshape(8192, 512)
    t2 = target.reshape(8192, 512)
    w2 = weight.reshape(8192, 512)

    elem = pl.pallas_call(
        _tc_elem,
        grid=(16,),
        in_specs=[pl.BlockSpec((512, 512), lambda i: (i, 0))] * 3,
        out_specs=pl.BlockSpec((2048, 128), lambda i: (i, 0)),
        out_shape=jax.ShapeDtypeStruct((32768, 128), f32),
    )
    u = elem(p2, t2, w2).reshape(N)

    cnt0, sum0 = _sc_hist(0)(u)
    m1 = pl.pallas_call(
        _tc_m1,
        out_shape=(jax.ShapeDtypeStruct((8, 128), jnp.int32),
                   jax.ShapeDtypeStruct((8, 128), f32)),
    )
    sel0, st1 = m1(cnt0.reshape(NW * 256, 128), sum0.reshape(NW * 256, 128))

    cnt1, sum1 = _sc_hist(1)(u, sel0.reshape(1024))
    m2 = pl.pallas_call(
        _tc_m2,
        out_shape=(jax.ShapeDtypeStruct((8, 128), jnp.int32),
                   jax.ShapeDtypeStruct((8, 128), f32)),
    )
    sel1, st2 = m2(cnt1.reshape(NW * 256, 128), sum1.reshape(NW * 256, 128),
                   sel0, st1)

    cnt2 = _sc_hist(2)(u, sel1.reshape(1024))
    m3 = pl.pallas_call(
        _tc_m3,
        out_shape=jax.ShapeDtypeStruct((1, 1), f32),
    )
    ans = m3(cnt2.reshape(NW * 128, 128), sel1, st2)
    return ans[0, 0]


# speedup vs baseline: 1.7762x; 1.7762x over previous
import functools

import jax
import jax.numpy as jnp
from jax import lax
from jax.experimental import pallas as pl
from jax.experimental.pallas import tpu as pltpu
from jax.experimental.pallas import tpu_sc as plsc

N = 1 << 22
K = 100000
NW = 32
PW = N // NW
CHUNK = 16384
NCHUNK = PW // CHUNK
VPC = CHUNK // 16
NB0 = 2048
NB1 = 2048
NB2 = 1024
SCALE = float(K) * float(N)


def _sc_mesh():
    return plsc.VectorSubcoreMesh(core_axis_name="c", subcore_axis_name="s")


def _zero_hist(ref, nwords):
    zeros = jnp.zeros((16,), jnp.float32)

    @plsc.parallel_loop(0, nwords // 16, unroll=8)
    def _(i):
        ref[pl.ds(i * 16, 16)] = zeros


def _hist_body(level):
    nb = (NB0, NB1, NB2)[level]
    sums = level < 2

    def body(*args):
        if level == 0:
            u_hbm = args[0]
            sel_hbm = None
            rest = args[1:]
        else:
            u_hbm, sel_hbm = args[0], args[1]
            rest = args[2:]
        if sums:
            cnt_hbm, sum_hbm = rest[0], rest[1]
            uv0, uv1, selv, cntv, sumv, sem0, sem1 = rest[2:]
        else:
            cnt_hbm = rest[0]
            sum_hbm = None
            uv0, uv1, selv, cntv, sumv, sem0, sem1 = rest[1:]

        wid = lax.axis_index("c") * 16 + lax.axis_index("s")
        base = wid * PW
        li = lax.iota(jnp.int32, 16)
        ones = jnp.ones((16,), jnp.float32)

        _zero_hist(cntv, nb * 16)
        if sums:
            _zero_hist(sumv, nb * 16)
        if level > 0:
            pltpu.sync_copy(sel_hbm.at[pl.ds(0, 16)], selv)
            selvec = selv[...]

        sems = (sem0, sem1)
        bufs = (uv0, uv1)
        pltpu.async_copy(u_hbm.at[pl.ds(base, CHUNK)], bufs[0], sems[0])
        for ci in range(NCHUNK):
            cur = ci % 2
            if ci + 1 < NCHUNK:
                pltpu.async_copy(
                    u_hbm.at[pl.ds(base + (ci + 1) * CHUNK, CHUNK)],
                    bufs[(ci + 1) % 2], sems[(ci + 1) % 2])
            pltpu.make_async_copy(
                u_hbm.at[pl.ds(base + ci * CHUNK, CHUNK)], bufs[cur],
                sems[cur]).wait()
            buf = bufs[cur]

            @plsc.parallel_loop(0, VPC, unroll=16)
            def _(vi):
                u = buf[pl.ds(vi * 16, 16)]
                bits = lax.bitcast_convert_type(u, jnp.int32)
                if level == 0:
                    flat = lax.shift_right_logical(bits, 21) * 16 + li
                    plsc.addupdate_scatter(cntv, [flat], ones)
                    plsc.addupdate_scatter(sumv, [flat], u)
                elif level == 1:
                    m = lax.shift_right_logical(bits, 21) == selvec
                    sub = lax.shift_right_logical(bits, 10) & 0x7FF
                    flat = sub * 16 + li
                    plsc.addupdate_scatter(cntv, [flat], ones, mask=m)
                    plsc.addupdate_scatter(sumv, [flat], u, mask=m)
                else:
                    m = lax.shift_right_logical(bits, 10) == selvec
                    flat = (bits & 0x3FF) * 16 + li
                    plsc.addupdate_scatter(cntv, [flat], ones, mask=m)

        pltpu.sync_copy(cntv, cnt_hbm.at[wid])
        if sums:
            pltpu.sync_copy(sumv, sum_hbm.at[wid])

    return body


def _sc_hist(level):
    nb = (NB0, NB1, NB2)[level]
    sums = level < 2
    f32 = jnp.float32
    hist = jax.ShapeDtypeStruct((NW, nb * 16), f32)
    out_type = (hist, hist) if sums else hist
    return pl.kernel(
        _hist_body(level),
        out_type=out_type,
        mesh=_sc_mesh(),
        compiler_params=pltpu.CompilerParams(needs_layout_passes=False),
        scratch_types=[
            pltpu.VMEM((CHUNK,), f32),
            pltpu.VMEM((CHUNK,), f32),
            pltpu.VMEM((16,), jnp.int32),
            pltpu.VMEM((nb * 16,), f32),
            pltpu.VMEM((nb * 16 if sums else 16,), f32),
            pltpu.SemaphoreType.DMA,
            pltpu.SemaphoreType.DMA,
        ],
    )


def _tc_elem(p_ref, t_ref, w_ref, u_ref):
    d = p_ref[...] - t_ref[...]
    u2 = w_ref[...] * d * d
    for j in range(4):
        u_ref[pl.ds(j * 512, 512), :] = u2[:, j * 128:(j + 1) * 128]


def _cell_select(cnt, kk):
    R, C = cnt.shape
    ci = lax.broadcasted_iota(jnp.int32, (C, C), 0)
    cj = lax.broadcasted_iota(jnp.int32, (C, C), 1)
    mg = ((ci // 16) >= (cj // 16)).astype(jnp.float32)
    gs = jnp.dot(cnt, mg, preferred_element_type=jnp.float32)
    rt = jnp.sum(cnt, axis=1, keepdims=True)
    ri = lax.broadcasted_iota(jnp.int32, (R, R), 0)
    rj = lax.broadcasted_iota(jnp.int32, (R, R), 1)
    mr = (rj > ri).astype(jnp.float32)
    sr = jnp.dot(mr, rt, preferred_element_type=jnp.float32)
    incl = sr + gs
    bidx = (lax.broadcasted_iota(jnp.int32, (R, C), 0) * 8
            + lax.broadcasted_iota(jnp.int32, (R, C), 1) // 16)
    sel = jnp.max(jnp.where(incl >= kk, bidx, -1))
    above = bidx > sel
    return sel, above, bidx


def _worker_sum(ref, rows):
    acc = ref[pl.ds(0, rows), :]
    for w in range(1, NW):
        acc = acc + ref[pl.ds(w * rows, rows), :]
    return acc


def _scalar_at(ref_val, r, c):
    row = lax.broadcasted_iota(jnp.int32, ref_val.shape, 0)
    col = lax.broadcasted_iota(jnp.int32, ref_val.shape, 1)
    zero = jnp.zeros((), ref_val.dtype)
    return jnp.sum(jnp.where((row == r) & (col == c), ref_val, zero))


def _bcast_rows(vals, dtype):
    out = jnp.zeros((8, 128), dtype)
    row = lax.broadcasted_iota(jnp.int32, (8, 128), 0)
    for r, v in enumerate(vals):
        out = jnp.where(row == r, v.astype(dtype), out)
    return out


def _tc_m1(cnt_ref, sum_ref, sel_out, st_out):
    cnt = _worker_sum(cnt_ref, 256)
    sm = _worker_sum(sum_ref, 256)
    kk = jnp.float32(K)
    sel, above, _ = _cell_select(cnt, kk)
    c_above = jnp.sum(jnp.where(above, cnt, 0.0))
    s_above = jnp.sum(jnp.where(above, sm, 0.0))
    sel_out[...] = jnp.full((8, 128), sel, jnp.int32)
    st_out[...] = _bcast_rows([kk - c_above, s_above], jnp.float32)


def _tc_m2(cnt_ref, sum_ref, sel_ref, st_ref, sel_out, st_out):
    cnt = _worker_sum(cnt_ref, 256)
    sm = _worker_sum(sum_ref, 256)
    b0 = _scalar_at(sel_ref[...], 0, 0)
    st = st_ref[...]
    k1 = _scalar_at(st, 0, 0)
    s0 = _scalar_at(st, 1, 0)
    sel, above, _ = _cell_select(cnt, k1)
    c_above = jnp.sum(jnp.where(above, cnt, 0.0))
    s_above = jnp.sum(jnp.where(above, sm, 0.0))
    prefix22 = b0 * NB1 + sel
    sel_out[...] = jnp.full((8, 128), prefix22, jnp.int32)
    st_out[...] = _bcast_rows([k1 - c_above, s0 + s_above], jnp.float32)


def _tc_m3(cnt_ref, sel_ref, st_ref, ans_out):
    cnt = _worker_sum(cnt_ref, 128)
    prefix22 = _scalar_at(sel_ref[...], 0, 0)
    st = st_ref[...]
    k2 = _scalar_at(st, 0, 0)
    s01 = _scalar_at(st, 1, 0)
    sel, above, bidx = _cell_select(cnt, k2)
    c_above = jnp.sum(jnp.where(above, cnt, 0.0))
    vals = lax.bitcast_convert_type(prefix22 * NB2 + bidx, jnp.float32)
    s2 = jnp.sum(jnp.where(above, cnt * vals, 0.0))
    t = lax.bitcast_convert_type(prefix22 * NB2 + sel, jnp.float32)
    ans = (s01 + s2 + (k2 - c_above) * t) / jnp.float32(SCALE)
    ans_out[...] = jnp.full((1, 1), ans, jnp.float32)


def kernel(predict, target, weight):
    f32 = jnp.float32
    p2 = predict.reshape(8192, 512)
    t2 = target.reshape(8192, 512)
    w2 = weight.reshape(8192, 512)

    elem = pl.pallas_call(
        _tc_elem,
        grid=(16,),
        in_specs=[pl.BlockSpec((512, 512), lambda i: (i, 0))] * 3,
        out_specs=pl.BlockSpec((2048, 128), lambda i: (i, 0)),
        out_shape=jax.ShapeDtypeStruct((32768, 128), f32),
    )
    u = elem(p2, t2, w2).reshape(N)

    cnt0, sum0 = _sc_hist(0)(u)
    m1 = pl.pallas_call(
        _tc_m1,
        out_shape=(jax.ShapeDtypeStruct((8, 128), jnp.int32),
                   jax.ShapeDtypeStruct((8, 128), f32)),
    )
    sel0, st1 = m1(cnt0.reshape(NW * 256, 128), sum0.reshape(NW * 256, 128))

    cnt1, sum1 = _sc_hist(1)(u, sel0.reshape(1024))
    m2 = pl.pallas_call(
        _tc_m2,
        out_shape=(jax.ShapeDtypeStruct((8, 128), jnp.int32),
                   jax.ShapeDtypeStruct((8, 128), f32)),
    )
    sel1, st2 = m2(cnt1.reshape(NW * 256, 128), sum1.reshape(NW * 256, 128),
                   sel0, st1)

    cnt2 = _sc_hist(2)(u, sel1.reshape(1024))
    m3 = pl.pallas_call(
        _tc_m3,
        out_shape=jax.ShapeDtypeStruct((1, 1), f32),
    )
    ans = m3(cnt2.reshape(NW * 128, 128), sel1, st2)
    return ans[0, 0]
